# baseline (speedup 1.0000x reference)
import jax
import jax.numpy as jnp
from jax import lax
from jax.experimental import pallas as pl
from jax.experimental.pallas import tpu as pltpu
from jax.experimental.pallas import tpu_sc as plsc

_MARGIN_S = 64.0
_MARGIN_M = 0.35
_DELTA = -(_MARGIN_S * _MARGIN_M)

_NC = 2
_NS = 16
_NW = _NC * _NS

_B = 1024
_N = 100000
_RW = 16000
_NR = (_B * _N) // _RW
_RPW = _NR // _NW
_LPW = _B // _NW
_LANES = 16
_UNROLL = 8


def _scale_row(src, dst):
    def body(i, _):
        for u in range(_UNROLL):
            sl = pl.ds((i * _UNROLL + u) * _LANES, _LANES)
            dst[sl] = src[sl] * _MARGIN_S
        return 0

    lax.fori_loop(0, _RW // (_LANES * _UNROLL), body, 0)


def _patch_row(dst, vlr, voff, cur_row):
    cur = jnp.full((_LANES,), cur_row, jnp.int32)
    for g in range(_LPW // _LANES):
        sl = pl.ds(g * _LANES, _LANES)
        m = vlr[sl] == cur
        offv = voff[sl]
        vals = plsc.load_gather(dst, [offv], mask=m)
        plsc.store_scatter(dst, [offv], vals + jnp.float32(_DELTA), mask=m)


def _sc_body(x_ref, lr_hbm, off_hbm, out_ref,
             in0, in1, ou0, ou1, vlr, voff,
             si0, si1, so0, so1):
    wid = lax.axis_index("s") * _NC + lax.axis_index("c")
    base = wid * _RPW

    pltpu.sync_copy(lr_hbm.at[pl.ds(wid * _LPW, _LPW)], vlr)
    pltpu.sync_copy(off_hbm.at[pl.ds(wid * _LPW, _LPW)], voff)

    ins = (in0, in1)
    outs = (ou0, ou1)
    isems = (si0, si1)
    osems = (so0, so1)

    def in_copy(row, b):
        return pltpu.make_async_copy(
            x_ref.at[pl.ds(row * _RW, _RW)], ins[b], isems[b])

    def out_copy(row, b):
        return pltpu.make_async_copy(
            outs[b], out_ref.at[pl.ds(row * _RW, _RW)], osems[b])

    for b in range(2):
        in_copy(base + b, b).start()

    def round_(i, _):
        r0 = i * 2
        for b in range(2):
            r = r0 + b
            row = base + r
            in_copy(row, b).wait()

            @pl.when(r >= 2)
            def _wait_prev_out():
                out_copy(row - 2, b).wait()

            _scale_row(ins[b], outs[b])
            _patch_row(outs[b], vlr, voff, row)
            out_copy(row, b).start()

            @pl.when(r + 2 < _RPW)
            def _next_in():
                in_copy(row + 2, b).start()

        return 0

    lax.fori_loop(0, _RPW // 2, round_, 0)

    for b in range(2):
        out_copy(base + _RPW - 2 + b, b).wait()


def _sc_margin(x1d, lr, off):
    mesh = plsc.VectorSubcoreMesh(core_axis_name="c", subcore_axis_name="s")
    return pl.kernel(
        _sc_body,
        out_type=jax.ShapeDtypeStruct((_B * _N,), jnp.float32),
        mesh=mesh,
        scratch_types=[
            pltpu.VMEM((_RW,), jnp.float32),
            pltpu.VMEM((_RW,), jnp.float32),
            pltpu.VMEM((_RW,), jnp.float32),
            pltpu.VMEM((_RW,), jnp.float32),
            pltpu.VMEM((_LPW,), jnp.int32),
            pltpu.VMEM((_LPW,), jnp.int32),
            pltpu.SemaphoreType.DMA,
            pltpu.SemaphoreType.DMA,
            pltpu.SemaphoreType.DMA,
            pltpu.SemaphoreType.DMA,
        ],
        compiler_params=pltpu.CompilerParams(needs_layout_passes=False),
    )(x1d, lr, off)


def kernel(orin_out, labels):
    b, n = orin_out.shape
    x1d = orin_out.reshape(b * n)
    k = jnp.arange(b, dtype=jnp.int32)
    p = k * n + labels.astype(jnp.int32)
    lr = p // _RW
    off = p % _RW
    out = _sc_margin(x1d, lr, off)
    return out.reshape(b, n)


# speedup vs baseline: 1.9913x; 1.9913x over previous
import jax
import jax.numpy as jnp
from jax import lax
from jax.experimental import pallas as pl
from jax.experimental.pallas import tpu as pltpu
from jax.experimental.pallas import tpu_sc as plsc

_MARGIN_S = 64.0
_MARGIN_M = 0.35
_DELTA = -(_MARGIN_S * _MARGIN_M)

_NC = 2
_NS = 16
_NW = _NC * _NS

_B = 1024
_N = 100000
_SUB = 8
_RGPW = (_B // _SUB) // _NW
_LPW = _B // _NW
_CW = 2048
_NCH = _N // _CW
_TW = 1664
_SLIV = _NCH * _CW + _TW
_CPW = _RGPW * _NCH
_LANES = 16
_UNROLL = 8


def _scale_chunk(src, dst, width):
    def body(i, _):
        for u in range(_UNROLL):
            sl = pl.ds((i * _UNROLL + u) * _LANES, _LANES)
            for s in range(_SUB):
                dst[s, sl] = src[s, sl] * _MARGIN_S
        return 0

    lax.fori_loop(0, width // (_LANES * _UNROLL), body, 0)


def _patch_chunk(dst, vcid, vsub, voff, cur_cid):
    cur = jnp.full((_LANES,), cur_cid, jnp.int32)
    for g in range(_LPW // _LANES):
        sl = pl.ds(g * _LANES, _LANES)
        m = vcid[sl] == cur
        subv = vsub[sl]
        offv = voff[sl]
        vals = plsc.load_gather(dst, [subv, offv], mask=m)
        plsc.store_scatter(dst, [subv, offv],
                           vals + jnp.float32(_DELTA), mask=m)


def _sc_body(x_ref, mcid_hbm, tcid_hbm, sub_hbm, off_hbm, out_ref,
             in0, in1, ou0, ou1, vmcid, vtcid, vsub, voff,
             si0, si1, so0, so1):
    wid = lax.axis_index("s") * _NC + lax.axis_index("c")
    rg0 = wid * _RGPW

    lsl = pl.ds(wid * _LPW, _LPW)
    pltpu.sync_copy(mcid_hbm.at[lsl], vmcid)
    pltpu.sync_copy(tcid_hbm.at[lsl], vtcid)
    pltpu.sync_copy(sub_hbm.at[lsl], vsub)
    pltpu.sync_copy(off_hbm.at[lsl], voff)

    ins = (in0, in1)
    outs = (ou0, ou1)
    isems = (si0, si1)
    osems = (so0, so1)

    def rows_of(cid):
        return pl.ds((rg0 + cid // _NCH) * _SUB, _SUB)

    def cols_of(cid):
        return pl.ds((cid % _NCH) * _CW, _CW)

    def in_copy(cid, b):
        return pltpu.make_async_copy(
            x_ref.at[rows_of(cid), cols_of(cid)], ins[b], isems[b])

    def out_copy(cid, b):
        return pltpu.make_async_copy(
            outs[b], out_ref.at[rows_of(cid), cols_of(cid)], osems[b])

    for b in range(2):
        in_copy(b, b).start()

    def round_(i, _):
        c0 = i * 2
        for b in range(2):
            cid = c0 + b
            in_copy(cid, b).wait()

            @pl.when(cid >= 2)
            def _wait_prev_out():
                out_copy(cid - 2, b).wait()

            _scale_chunk(ins[b], outs[b], _CW)
            _patch_chunk(outs[b], vmcid, vsub, voff, cid)
            out_copy(cid, b).start()

            @pl.when(cid + 2 < _CPW)
            def _next_in():
                in_copy(cid + 2, b).start()

        return 0

    lax.fori_loop(0, _CPW // 2, round_, 0)

    for b in range(2):
        out_copy(_CPW - 2 + b, b).wait()

    tsl = pl.ds(0, _TW)
    for t in range(_RGPW):
        rsl = pl.ds((rg0 + t) * _SUB, _SUB)
        csl = pl.ds(_NCH * _CW, _TW)
        pltpu.sync_copy(x_ref.at[rsl, csl], in0.at[:, tsl])
        _scale_chunk(in0, ou0, _TW)
        _patch_chunk(ou0, vtcid, vsub, voff, t)
        pltpu.sync_copy(ou0.at[:, tsl], out_ref.at[rsl, csl])


def _sc_margin(x, mcid, tcid, sub, off):
    mesh = plsc.VectorSubcoreMesh(core_axis_name="c", subcore_axis_name="s")
    return pl.kernel(
        _sc_body,
        out_type=jax.ShapeDtypeStruct((_B, _N), jnp.float32),
        mesh=mesh,
        scratch_types=[
            pltpu.VMEM((_SUB, _CW), jnp.float32),
            pltpu.VMEM((_SUB, _CW), jnp.float32),
            pltpu.VMEM((_SUB, _CW), jnp.float32),
            pltpu.VMEM((_SUB, _CW), jnp.float32),
            pltpu.VMEM((_LPW,), jnp.int32),
            pltpu.VMEM((_LPW,), jnp.int32),
            pltpu.VMEM((_LPW,), jnp.int32),
            pltpu.VMEM((_LPW,), jnp.int32),
            pltpu.SemaphoreType.DMA,
            pltpu.SemaphoreType.DMA,
            pltpu.SemaphoreType.DMA,
            pltpu.SemaphoreType.DMA,
        ],
        compiler_params=pltpu.CompilerParams(
            needs_layout_passes=False, use_tc_tiling_on_sc=True),
    )(x, mcid, tcid, sub, off)


_SLW = 128


def _sliver_block(labels_ref, x_ref, prev_ref, o_ref):
    x = x_ref[...]
    labels = labels_ref[...]
    cols = _SLIV + jax.lax.broadcasted_iota(jnp.int32, x.shape, 1)
    mask = cols == labels
    o_ref[...] = x * _MARGIN_S - jnp.where(mask, _MARGIN_S * _MARGIN_M, 0.0)


def _sliver_fix(x, prev, labels2d):
    return pl.pallas_call(
        _sliver_block,
        grid=(1,),
        in_specs=[
            pl.BlockSpec((_B, 1), lambda i: (0, 0)),
            pl.BlockSpec((_B, _SLW), lambda i: (0, _SLIV // _SLW)),
            pl.BlockSpec(memory_space=pltpu.MemorySpace.HBM),
        ],
        out_specs=pl.BlockSpec((_B, _SLW), lambda i: (0, _SLIV // _SLW)),
        out_shape=jax.ShapeDtypeStruct((_B, _N), jnp.float32),
        input_output_aliases={2: 0},
    )(labels2d, x, prev)


def kernel(orin_out, labels):
    b, n = orin_out.shape
    lab = labels.astype(jnp.int32)
    r = jnp.arange(b, dtype=jnp.int32)
    rg_local = (r // _SUB) % _RGPW
    ch = lab // _CW
    is_tail = jnp.logical_and(ch >= _NCH, lab < _SLIV)
    is_sliver = lab >= _SLIV
    mcid = jnp.where(jnp.logical_or(is_tail, is_sliver),
                     -1, rg_local * _NCH + ch)
    tcid = jnp.where(is_tail, rg_local, -1)
    sub = r % _SUB
    off = jnp.where(is_tail, lab - _NCH * _CW, lab % _CW)
    out = _sc_margin(orin_out, mcid, tcid, sub, off)
    return _sliver_fix(orin_out, out, lab.reshape(b, 1))


# speedup vs baseline: 1.9984x; 1.0035x over previous
import jax
import jax.numpy as jnp
from jax import lax
from jax.experimental import pallas as pl
from jax.experimental.pallas import tpu as pltpu
from jax.experimental.pallas import tpu_sc as plsc

_MARGIN_S = 64.0
_MARGIN_M = 0.35
_DELTA = -(_MARGIN_S * _MARGIN_M)

_NC = 2
_NS = 16
_NW = _NC * _NS

_B = 1024
_N = 100000
_SUB = 8
_RGPW = (_B // _SUB) // _NW
_LPW = _B // _NW
_CW = 2048
_NCH = _N // _CW
_TW = 1664
_SLIV = _NCH * _CW + _TW
_CPW = _RGPW * _NCH
_LANES = 16
_UNROLL = 8


def _scale_chunk(src, dst, width):
    def body(i, _):
        for u in range(_UNROLL):
            sl = pl.ds((i * _UNROLL + u) * _LANES, _LANES)
            for s in range(_SUB):
                dst[s, sl] = src[s, sl] * _MARGIN_S
        return 0

    lax.fori_loop(0, width // (_LANES * _UNROLL), body, 0)


def _patch_chunk(dst, vcid, vsub, voff, cur_cid):
    cur = jnp.full((_LANES,), cur_cid, jnp.int32)
    for g in range(_LPW // _LANES):
        sl = pl.ds(g * _LANES, _LANES)
        m = vcid[sl] == cur
        subv = vsub[sl]
        offv = voff[sl]
        vals = plsc.load_gather(dst, [subv, offv], mask=m)
        plsc.store_scatter(dst, [subv, offv],
                           vals + jnp.float32(_DELTA), mask=m)


def _sc_body(x_ref, mcid_hbm, tcid_hbm, sub_hbm, off_hbm, out_ref,
             in0, in1, in2, ou0, ou1, ou2, vmcid, vtcid, vsub, voff,
             si0, si1, si2, so0, so1, so2):
    wid = lax.axis_index("s") * _NC + lax.axis_index("c")
    rg0 = wid * _RGPW

    lsl = pl.ds(wid * _LPW, _LPW)
    pltpu.sync_copy(mcid_hbm.at[lsl], vmcid)
    pltpu.sync_copy(tcid_hbm.at[lsl], vtcid)
    pltpu.sync_copy(sub_hbm.at[lsl], vsub)
    pltpu.sync_copy(off_hbm.at[lsl], voff)

    ins = (in0, in1, in2)
    outs = (ou0, ou1, ou2)
    isems = (si0, si1, si2)
    osems = (so0, so1, so2)

    def rows_of(cid):
        return pl.ds((rg0 + cid // _NCH) * _SUB, _SUB)

    def cols_of(cid):
        return pl.ds((cid % _NCH) * _CW, _CW)

    def in_copy(cid, b):
        return pltpu.make_async_copy(
            x_ref.at[rows_of(cid), cols_of(cid)], ins[b], isems[b])

    def out_copy(cid, b):
        return pltpu.make_async_copy(
            outs[b], out_ref.at[rows_of(cid), cols_of(cid)], osems[b])

    for b in range(_NBUF):
        in_copy(b, b).start()

    def round_(i, _):
        c0 = i * _NBUF
        for b in range(_NBUF):
            cid = c0 + b
            in_copy(cid, b).wait()

            @pl.when(cid >= _NBUF)
            def _wait_prev_out():
                out_copy(cid - _NBUF, b).wait()

            _scale_chunk(ins[b], outs[b], _CW)
            _patch_chunk(outs[b], vmcid, vsub, voff, cid)
            out_copy(cid, b).start()

            @pl.when(cid + _NBUF < _CPW)
            def _next_in():
                in_copy(cid + _NBUF, b).start()

        return 0

    lax.fori_loop(0, _CPW // _NBUF, round_, 0)

    for b in range(_NBUF):
        out_copy(_CPW - _NBUF + b, b).wait()

    tsl = pl.ds(0, _TW)
    for t in range(_RGPW):
        rsl = pl.ds((rg0 + t) * _SUB, _SUB)
        csl = pl.ds(_NCH * _CW, _TW)
        pltpu.sync_copy(x_ref.at[rsl, csl], in0.at[:, tsl])
        _scale_chunk(in0, ou0, _TW)
        _patch_chunk(ou0, vtcid, vsub, voff, t)
        pltpu.sync_copy(ou0.at[:, tsl], out_ref.at[rsl, csl])


def _sc_margin(x, mcid, tcid, sub, off):
    mesh = plsc.VectorSubcoreMesh(core_axis_name="c", subcore_axis_name="s")
    return pl.kernel(
        _sc_body,
        out_type=jax.ShapeDtypeStruct((_B, _N), jnp.float32),
        mesh=mesh,
        scratch_types=(
            [pltpu.VMEM((_SUB, _CW), jnp.float32)] * (2 * _NBUF)
            + [pltpu.VMEM((_LPW,), jnp.int32)] * 4
            + [pltpu.SemaphoreType.DMA] * (2 * _NBUF)
        ),
        compiler_params=pltpu.CompilerParams(
            needs_layout_passes=False, use_tc_tiling_on_sc=True),
    )(x, mcid, tcid, sub, off)


_SLW = 128
_NBUF = 3


def _sliver_block(labels_ref, x_ref, prev_ref, o_ref):
    x = x_ref[...]
    labels = labels_ref[...]
    cols = _SLIV + jax.lax.broadcasted_iota(jnp.int32, x.shape, 1)
    mask = cols == labels
    o_ref[...] = x * _MARGIN_S - jnp.where(mask, _MARGIN_S * _MARGIN_M, 0.0)


def _sliver_fix(x, prev, labels2d):
    return pl.pallas_call(
        _sliver_block,
        grid=(1,),
        in_specs=[
            pl.BlockSpec((_B, 1), lambda i: (0, 0)),
            pl.BlockSpec((_B, _SLW), lambda i: (0, _SLIV // _SLW)),
            pl.BlockSpec(memory_space=pltpu.MemorySpace.HBM),
        ],
        out_specs=pl.BlockSpec((_B, _SLW), lambda i: (0, _SLIV // _SLW)),
        out_shape=jax.ShapeDtypeStruct((_B, _N), jnp.float32),
        input_output_aliases={2: 0},
    )(labels2d, x, prev)


def kernel(orin_out, labels):
    b, n = orin_out.shape
    lab = labels.astype(jnp.int32)
    r = jnp.arange(b, dtype=jnp.int32)
    rg_local = (r // _SUB) % _RGPW
    ch = lab // _CW
    is_tail = jnp.logical_and(ch >= _NCH, lab < _SLIV)
    is_sliver = lab >= _SLIV
    mcid = jnp.where(jnp.logical_or(is_tail, is_sliver),
                     -1, rg_local * _NCH + ch)
    tcid = jnp.where(is_tail, rg_local, -1)
    sub = r % _SUB
    off = jnp.where(is_tail, lab - _NCH * _CW, lab % _CW)
    out = _sc_margin(orin_out, mcid, tcid, sub, off)
    return _sliver_fix(orin_out, out, lab.reshape(b, 1))
